# baseline (speedup 1.0000x reference)
import functools

import jax
import jax.numpy as jnp
from jax import lax
from jax.experimental import pallas as pl
from jax.experimental.pallas import tpu as pltpu
from jax.experimental.pallas import tpu_sc as plsc

MASKED = -1000000000000.0

_NC = 2
_NS = 16


def _sc_place_builder(v, b, k):
    mesh = plsc.VectorSubcoreMesh(core_axis_name="c", subcore_axis_name="s")

    @functools.partial(
        pl.kernel,
        mesh=mesh,
        out_type=(),
        scratch_types=[
            [pltpu.VMEM((128,), jnp.int32) for _ in range(2 * k)],
            [pltpu.VMEM((128, b), jnp.float32) for _ in range(2 * k)],
            pltpu.SemaphoreType.DMA,
            pltpu.SemaphoreType.DMA,
        ],
        compiler_params=pltpu.CompilerParams(needs_layout_passes=False),
    )
    def sc_place(vlog, nlog, vids2, nids2, vout, nout, idx_refs, row_refs, gsem, ssem):
        c = lax.axis_index("c")
        s = lax.axis_index("s")
        w = s * _NC + c

        def go(log_hbm, ids_hbm, out_ref, idxs, rows):
            for j in range(k):
                pltpu.sync_copy(ids_hbm.at[w * k + j, 0], idxs[j])
            gathers = []
            for j in range(k):
                gathers.append(
                    pltpu.async_copy(log_hbm.at[idxs[j]], rows[j], gsem)
                )
            for cp in gathers:
                cp.wait()
            scatters = []
            for j in range(k):
                scatters.append(
                    pltpu.async_copy(rows[j], out_ref.at[idxs[j]], ssem)
                )
            for cp in scatters:
                cp.wait()

        go(vlog, vids2, vout, idx_refs[:k], row_refs[:k])
        go(nlog, nids2, nout, idx_refs[k:], row_refs[k:])

    return sc_place


def _fill_body(vout_ref, nout_ref):
    vout_ref[...] = jnp.full(vout_ref.shape, MASKED, jnp.float32)
    nout_ref[...] = jnp.full(nout_ref.shape, MASKED, jnp.float32)


def kernel(verb_logits, noun_logits, seen_verb_ids, seen_noun_ids):
    b, v = verb_logits.shape
    n = seen_verb_ids.shape[0]

    vlog_t = verb_logits.T
    nlog_t = noun_logits.T

    k = -(-n // (_NS * _NC * 128))
    n_pad = _NS * _NC * 128 * k
    if n_pad != n:
        pad_v = jnp.broadcast_to(seen_verb_ids[:1], (n_pad - n,))
        pad_n = jnp.broadcast_to(seen_noun_ids[:1], (n_pad - n,))
        vids = jnp.concatenate([seen_verb_ids, pad_v])
        nids = jnp.concatenate([seen_noun_ids, pad_n])
    else:
        vids, nids = seen_verb_ids, seen_noun_ids
    vids2 = vids.reshape(_NS * _NC * k, 1, 128)
    nids2 = nids.reshape(_NS * _NC * k, 1, 128)

    rows = 10240
    grid = (v + rows - 1) // rows
    filled = pl.pallas_call(
        _fill_body,
        grid=(grid,),
        in_specs=[],
        out_specs=[
            pl.BlockSpec((rows, b), lambda i: (i, 0)),
            pl.BlockSpec((rows, b), lambda i: (i, 0)),
        ],
        out_shape=[
            jax.ShapeDtypeStruct((v, b), jnp.float32),
            jax.ShapeDtypeStruct((v, b), jnp.float32),
        ],
        compiler_params=pltpu.CompilerParams(
            dimension_semantics=("parallel",),
        ),
    )()

    vout_ref = jax.new_ref(filled[0])
    nout_ref = jax.new_ref(filled[1])
    _sc_place_builder(v, b, k)(vlog_t, nlog_t, vids2, nids2, vout_ref, nout_ref)

    return (vout_ref[...].T, nout_ref[...].T)


# speedup vs baseline: 4.4729x; 4.4729x over previous
import functools

import jax
import jax.numpy as jnp
from jax import lax
from jax.experimental import pallas as pl
from jax.experimental.pallas import tpu as pltpu
from jax.experimental.pallas import tpu_sc as plsc

MASKED = -1000000000000.0

_NC = 2
_NS = 16
_LANES = 16


def _sc_mask_builder(v_pad, n, chunk):
    mesh = plsc.VectorSubcoreMesh(core_axis_name="c", subcore_axis_name="s")

    @functools.partial(
        pl.kernel,
        mesh=mesh,
        out_type=(
            jax.ShapeDtypeStruct((v_pad,), jnp.float32),
            jax.ShapeDtypeStruct((v_pad,), jnp.float32),
        ),
        scratch_types=[
            pltpu.VMEM((n,), jnp.int32),
            pltpu.VMEM((chunk,), jnp.float32),
            pltpu.SemaphoreType.DMA,
        ],
        compiler_params=pltpu.CompilerParams(needs_layout_passes=False),
    )
    def sc_mask(vids_hbm, nids_hbm, vmask_hbm, nmask_hbm, ids_v, chunk_v, sem):
        c = lax.axis_index("c")
        s = lax.axis_index("s")

        zeros16 = jnp.zeros((_LANES,), jnp.float32)
        ones16 = jnp.ones((_LANES,), jnp.float32)
        n_full = n // _LANES
        tail = n % _LANES
        base = pl.multiple_of(s * chunk, 8)

        def build(ids_hbm, mask_hbm):
            ids_cp = pltpu.async_copy(ids_hbm, ids_v, sem)

            def zero_body(i, _):
                chunk_v[pl.ds(i * _LANES, _LANES)] = zeros16
                return 0

            lax.fori_loop(0, chunk // _LANES, zero_body, 0)
            ids_cp.wait()

            def scatter_at(off):
                ids16 = ids_v[pl.ds(off, _LANES)]
                local = ids16 - base
                in_range = (local >= 0) & (local < chunk)
                safe = jnp.where(in_range, local, 0)
                plsc.store_scatter(chunk_v, [safe], ones16, mask=in_range)

            def scatter_body(j, _):
                scatter_at(j * _LANES)
                return 0

            lax.fori_loop(0, n_full, scatter_body, 0)
            if tail:
                scatter_at(n - _LANES)

            pltpu.sync_copy(chunk_v, mask_hbm.at[pl.ds(base, chunk)])

        @pl.when(c == 0)
        def _():
            build(vids_hbm, vmask_hbm)

        @pl.when(c == 1)
        def _():
            build(nids_hbm, nmask_hbm)

    return sc_mask


def _tc_select_body(vmask_ref, nmask_ref, vlog_ref, nlog_ref, vout_ref, nout_ref):
    b = vlog_ref.shape[1]
    rows = vmask_ref.shape[0]
    ones_row = jnp.ones((1, b), jnp.float32)
    dn = (((0,), (0,)), ((), ()))
    vb = jax.lax.dot_general(vmask_ref[...].reshape(1, rows), ones_row, dn,
                             preferred_element_type=jnp.float32)
    nb = jax.lax.dot_general(nmask_ref[...].reshape(1, rows), ones_row, dn,
                             preferred_element_type=jnp.float32)
    vout_ref[...] = jnp.where(vb != 0.0, vlog_ref[...], MASKED)
    nout_ref[...] = jnp.where(nb != 0.0, nlog_ref[...], MASKED)


def kernel(verb_logits, noun_logits, seen_verb_ids, seen_noun_ids):
    b, v = verb_logits.shape
    n = seen_verb_ids.shape[0]

    rows = 10240
    grid = (v + rows - 1) // rows
    v_pad = grid * rows
    chunk = v_pad // _NS
    assert chunk % 8 == 0

    vmask, nmask = _sc_mask_builder(v_pad, n, chunk)(seen_verb_ids, seen_noun_ids)

    vlog_t = verb_logits.T
    nlog_t = noun_logits.T

    out = pl.pallas_call(
        _tc_select_body,
        grid=(grid,),
        in_specs=[
            pl.BlockSpec((rows,), lambda i: (i,)),
            pl.BlockSpec((rows,), lambda i: (i,)),
            pl.BlockSpec((rows, b), lambda i: (i, 0)),
            pl.BlockSpec((rows, b), lambda i: (i, 0)),
        ],
        out_specs=[
            pl.BlockSpec((rows, b), lambda i: (i, 0)),
            pl.BlockSpec((rows, b), lambda i: (i, 0)),
        ],
        out_shape=[
            jax.ShapeDtypeStruct((v, b), jnp.float32),
            jax.ShapeDtypeStruct((v, b), jnp.float32),
        ],
        compiler_params=pltpu.CompilerParams(
            dimension_semantics=("parallel",),
        ),
    )(vmask, nmask, vlog_t, nlog_t)

    return (out[0].T, out[1].T)
